# baseline (speedup 1.0000x reference)
import jax
import jax.numpy as jnp
from jax import lax
from jax.experimental import pallas as pl
from jax.experimental.pallas import tpu as pltpu, tpu_sc as plsc

B, K, A, V = 4096, 26, 100, 128
NC, NS = 2, 16
NW = NC * NS
SPW = B // NW
LANES = 16
RING = 4


def _body(values_hbm, idx_hbm, out_hbm, idx_v, zero_v, vals_v,
          sz0, sz1, sz2, sz3, ss0, ss1, ss2, ss3, si0, si1, si2, si3):
    c = lax.axis_index("c")
    s = lax.axis_index("s")
    wid = s * NC + c
    base = wid * SPW

    pltpu.sync_copy(idx_hbm.at[pl.ds(base, SPW)], idx_v)

    zero16 = jnp.zeros((LANES,), jnp.float32)

    def zrow(a, acc):
        for j in range(V // LANES):
            zero_v[a, pl.ds(j * LANES, LANES)] = zero16
        return acc
    lax.fori_loop(0, A, zrow, 0)

    sem_z = (sz0, sz1, sz2, sz3)
    sem_s = (ss0, ss1, ss2, ss3)
    sem_in = (si0, si1, si2, si3)

    def z_cp(r, b):
        return pltpu.make_async_copy(zero_v, out_hbm.at[b], sem_z[r])

    def in_cp(r, b):
        return pltpu.make_async_copy(values_hbm.at[b], vals_v.at[r], sem_in[r])

    def s_cp(r, b, si):
        return pltpu.make_async_copy(
            vals_v.at[r], out_hbm.at[b].at[idx_v.at[si]], sem_s[r])

    in_cp(0, base).start()
    in_cp(1, base + 1).start()
    for r in range(RING):
        z_cp(r, base + r).start()

    def step(g, carry):
        for r in range(RING):
            si = RING * g + r
            b = base + si
            rp = (r + 2) % RING

            z_cp(r, b).wait()

            @pl.when(si >= 2)
            def _():
                s_cp(rp, b - 2, si - 2).wait()

            in_cp(r, b).wait()

            iv0 = idx_v[si, pl.ds(0, LANES)]
            iv1 = idx_v[si, pl.ds(K - LANES, LANES)]
            w0 = jnp.zeros((LANES,), jnp.int32)
            w1 = jnp.zeros((LANES,), jnp.int32)
            idx_sc = [iv0[k] if k < LANES else iv1[k - (K - LANES)]
                      for k in range(K)]
            for kp in range(K):
                kv = jnp.full((LANES,), kp, jnp.int32)
                w0 = jnp.where(iv0 == idx_sc[kp], kv, w0)
                w1 = jnp.where(iv1 == idx_sc[kp], kv, w1)
            for k in range(K):
                w = w0[k] if k < LANES else w1[k - (K - LANES)]

                @pl.when(w != k)
                def _():
                    for j in range(V // LANES):
                        vals_v[r, k, pl.ds(j * LANES, LANES)] = (
                            vals_v[r, w, pl.ds(j * LANES, LANES)])

            s_cp(r, b, si).start()

            @pl.when(si + 2 < SPW)
            def _():
                in_cp(rp, b + 2).start()

            @pl.when(si + RING < SPW)
            def _():
                z_cp(r, b + RING).start()
        return carry

    lax.fori_loop(0, SPW // RING, step, 0)

    s_cp((SPW - 2) % RING, base + SPW - 2, SPW - 2).wait()
    s_cp((SPW - 1) % RING, base + SPW - 1, SPW - 1).wait()


def kernel(values, attrib_idx):
    idx32 = attrib_idx.astype(jnp.int32)
    mesh = plsc.VectorSubcoreMesh(core_axis_name="c", subcore_axis_name="s")
    run = pl.kernel(
        _body,
        out_type=jax.ShapeDtypeStruct((B, A, V), jnp.float32),
        mesh=mesh,
        scratch_types=[
            pltpu.VMEM((SPW, K), jnp.int32),
            pltpu.VMEM((A, V), jnp.float32),
            pltpu.VMEM((RING, K, V), jnp.float32),
        ] + [pltpu.SemaphoreType.DMA] * 12,
    )
    return run(values, idx32)
